# baseline (speedup 1.0000x reference)
import functools
import math

import jax
import jax.numpy as jnp
from jax import lax
from jax.experimental import pallas as pl
from jax.experimental.pallas import tpu as pltpu
from jax.experimental.pallas import tpu_sc as plsc

_MAX_TRAIN_YEAR = 50
_MAX_LEN = 1000


def _pe_table(max_len, d_model):
    position = jnp.arange(0, max_len, dtype=jnp.float32)[:, None]
    div_term = jnp.exp(
        jnp.arange(0, d_model, 2, dtype=jnp.float32)
        * (-math.log(10000.0) / d_model)
    )
    pe = jnp.zeros((max_len, d_model), dtype=jnp.float32)
    pe = pe.at[:, 0::2].set(jnp.sin(position * div_term))
    pe = pe.at[:, 1::2].set(jnp.cos(position * div_term))
    return pe


def _make_sc_rowbias(BT, T, D, n_month, n_weekday, n_hour, n_year):
    mesh = plsc.VectorSubcoreMesh(core_axis_name="c", subcore_axis_name="s")
    NC = mesh.num_cores
    NS = mesh.num_subcores
    NW = NC * NS
    assert BT % NW == 0
    rows_per_w = BT // NW
    assert rows_per_w % 16 == 0
    L = 16

    @functools.partial(
        pl.kernel,
        out_type=jax.ShapeDtypeStruct((BT * D,), jnp.float32),
        mesh=mesh,
        compiler_params=pltpu.CompilerParams(needs_layout_passes=False),
        scratch_types=[
            pltpu.VMEM((rows_per_w * 4,), jnp.int32),
            pltpu.VMEM((n_month * D,), jnp.float32),
            pltpu.VMEM((n_weekday * D,), jnp.float32),
            pltpu.VMEM((n_hour * D,), jnp.float32),
            pltpu.VMEM((n_year * D,), jnp.float32),
            pltpu.VMEM((T * D,), jnp.float32),
            pltpu.VMEM((rows_per_w * D,), jnp.float32),
        ],
    )
    def sc_rowbias(
        xt_hbm, mo_hbm, wd_hbm, hr_hbm, yr_hbm, pe_hbm, out_hbm,
        idx_v, mo_v, wd_v, hr_v, yr_v, pe_v, out_v,
    ):
        wid = lax.axis_index("s") * NC + lax.axis_index("c")
        base = wid * rows_per_w
        pltpu.sync_copy(xt_hbm.at[pl.ds(base * 4, rows_per_w * 4)], idx_v)
        pltpu.sync_copy(mo_hbm, mo_v)
        pltpu.sync_copy(wd_hbm, wd_v)
        pltpu.sync_copy(hr_hbm, hr_v)
        pltpu.sync_copy(yr_hbm, yr_v)
        pltpu.sync_copy(pe_hbm, pe_v)

        lanes = lax.iota(jnp.int32, L)
        for g in range(rows_per_w // L):
            rows_loc = lanes + g * L
            i4 = rows_loc * 4
            m_off = plsc.load_gather(idx_v, [i4]) * D
            w_off = plsc.load_gather(idx_v, [i4 + 1]) * D
            h_off = plsc.load_gather(idx_v, [i4 + 2]) * D
            y_idx = jnp.minimum(plsc.load_gather(idx_v, [i4 + 3]), n_year - 1)
            y_off = y_idx * D
            t_off = jnp.remainder(rows_loc + base, T) * D
            r_off = rows_loc * D

            def dbody(d, _):
                dv = jnp.full((L,), d, jnp.int32)
                acc = plsc.load_gather(mo_v, [m_off + dv])
                acc = acc + plsc.load_gather(wd_v, [w_off + dv])
                acc = acc + plsc.load_gather(hr_v, [h_off + dv])
                acc = acc + plsc.load_gather(yr_v, [y_off + dv])
                acc = acc + plsc.load_gather(pe_v, [t_off + dv])
                plsc.store_scatter(out_v, [r_off + dv], acc)
                return 0

            lax.fori_loop(0, D, dbody, 0)

        pltpu.sync_copy(out_v, out_hbm.at[pl.ds(base * D, rows_per_w * D)])

    return sc_rowbias


def _tc_body(x_ref, w_ref, rb_ref, ent_ref, o_ref):
    Rt, A, F = x_ref.shape
    xb = x_ref[...].reshape(Rt * A, F)
    v = jnp.dot(xb, w_ref[...], preferred_element_type=jnp.float32)
    v = v.reshape(Rt, A, w_ref.shape[1])
    o_ref[...] = v + rb_ref[...][:, None, :] + ent_ref[...][None, :, :]


def _tc_fused(x3, W, rowbias, ent, Rt):
    BT, A, F = x3.shape
    D = W.shape[1]
    return pl.pallas_call(
        _tc_body,
        grid=(BT // Rt,),
        in_specs=[
            pl.BlockSpec((Rt, A, F), lambda i: (i, 0, 0)),
            pl.BlockSpec((F, D), lambda i: (0, 0)),
            pl.BlockSpec((Rt, D), lambda i: (i, 0)),
            pl.BlockSpec((A, D), lambda i: (0, 0)),
        ],
        out_specs=pl.BlockSpec((Rt, A, D), lambda i: (i, 0, 0)),
        out_shape=jax.ShapeDtypeStruct((BT, A, D), jnp.float32),
    )(x3, W, rowbias, ent)


def kernel(x, x_temp, W, b, t_month, t_weekday, t_hour, t_year, ent_table):
    B, T, A, F = x.shape
    D = W.shape[1]
    BT = B * T
    xt = x_temp.astype(jnp.int32).reshape(BT, 4)
    pe_b = _pe_table(_MAX_LEN, D)[:T] + b[None, :]

    sc_rowbias = _make_sc_rowbias(
        BT, T, D,
        t_month.shape[0], t_weekday.shape[0], t_hour.shape[0], t_year.shape[0],
    )
    rowbias = sc_rowbias(
        xt.reshape(-1), t_month.reshape(-1), t_weekday.reshape(-1),
        t_hour.reshape(-1), t_year.reshape(-1), pe_b.reshape(-1),
    ).reshape(BT, D)

    x3 = x.reshape(BT, A, F)
    out3 = _tc_fused(x3, W, rowbias, ent_table, Rt=32)
    return out3.reshape(B, T, A, D)


# speedup vs baseline: 2.1274x; 2.1274x over previous
import functools
import math

import jax
import jax.numpy as jnp
from jax import lax
from jax.experimental import pallas as pl
from jax.experimental.pallas import tpu as pltpu
from jax.experimental.pallas import tpu_sc as plsc

_MAX_LEN = 1000


def _pe_table(max_len, d_model):
    position = jnp.arange(0, max_len, dtype=jnp.float32)[:, None]
    div_term = jnp.exp(
        jnp.arange(0, d_model, 2, dtype=jnp.float32)
        * (-math.log(10000.0) / d_model)
    )
    pe = jnp.zeros((max_len, d_model), dtype=jnp.float32)
    pe = pe.at[:, 0::2].set(jnp.sin(position * div_term))
    pe = pe.at[:, 1::2].set(jnp.cos(position * div_term))
    return pe


def _make_sc_rowbias(BT, T, D, n_month, n_weekday, n_hour, n_year):
    mesh = plsc.VectorSubcoreMesh(core_axis_name="c", subcore_axis_name="s")
    NC = mesh.num_cores
    NS = mesh.num_subcores
    NW = NC * NS
    assert BT % NW == 0
    rows_per_w = BT // NW
    assert rows_per_w % 16 == 0
    assert T % rows_per_w == 0 or rows_per_w % T == 0
    L = 16

    @functools.partial(
        pl.kernel,
        out_type=jax.ShapeDtypeStruct((BT * D,), jnp.float32),
        mesh=mesh,
        compiler_params=pltpu.CompilerParams(needs_layout_passes=False),
        scratch_types=[
            pltpu.VMEM((rows_per_w * 4,), jnp.int32),
            pltpu.VMEM((n_month * D,), jnp.float32),
            pltpu.VMEM((n_weekday * D,), jnp.float32),
            pltpu.VMEM((n_hour * D,), jnp.float32),
            pltpu.VMEM((n_year * D,), jnp.float32),
            pltpu.VMEM((rows_per_w * D,), jnp.float32),
        ],
    )
    def sc_rowbias(
        xt_hbm, mo_hbm, wd_hbm, hr_hbm, yr_hbm, pe_hbm, out_hbm,
        idx_v, mo_v, wd_v, hr_v, yr_v, out_v,
    ):
        wid = lax.axis_index("s") * NC + lax.axis_index("c")
        base = wid * rows_per_w
        t0 = jnp.remainder(base, T)
        pltpu.sync_copy(xt_hbm.at[pl.ds(base * 4, rows_per_w * 4)], idx_v)
        pltpu.sync_copy(mo_hbm, mo_v)
        pltpu.sync_copy(wd_hbm, wd_v)
        pltpu.sync_copy(hr_hbm, hr_v)
        pltpu.sync_copy(yr_hbm, yr_v)
        pltpu.sync_copy(pe_hbm.at[pl.ds(t0 * D, rows_per_w * D)], out_v)

        lanes = lax.iota(jnp.int32, L)
        for g in range(rows_per_w // L):
            rows_loc = lanes + g * L
            i4 = rows_loc * 4
            m_off = plsc.load_gather(idx_v, [i4]) * D
            w_off = plsc.load_gather(idx_v, [i4 + 1]) * D
            h_off = plsc.load_gather(idx_v, [i4 + 2]) * D
            y_idx = jnp.minimum(plsc.load_gather(idx_v, [i4 + 3]), n_year - 1)
            y_off = y_idx * D
            r_off = rows_loc * D

            def dbody(k, _):
                for u in range(4):
                    dv = jnp.full((L,), 0, jnp.int32) + (k * 4 + u)
                    acc = plsc.load_gather(mo_v, [m_off + dv])
                    acc = acc + plsc.load_gather(wd_v, [w_off + dv])
                    acc = acc + plsc.load_gather(hr_v, [h_off + dv])
                    acc = acc + plsc.load_gather(yr_v, [y_off + dv])
                    plsc.addupdate_scatter(out_v, [r_off + dv], acc)
                return 0

            lax.fori_loop(0, D // 4, dbody, 0)

        pltpu.sync_copy(out_v, out_hbm.at[pl.ds(base * D, rows_per_w * D)])

    return sc_rowbias


def _tc_body(x_ref, w_ref, rb_ref, ent_ref, o_ref):
    _, Tt, F, A = x_ref.shape
    D = o_ref.shape[2]
    xm = x_ref[0].reshape(Tt * F, A)
    rbb = lax.broadcast_in_dim(rb_ref[0], (Tt, D, A), (0, 1))
    entb = ent_ref[...][None]
    G = 4 * F
    for j in range(Tt // 4):
        oj = jnp.dot(
            w_ref[...], xm[j * G:(j + 1) * G],
            preferred_element_type=jnp.float32,
        )
        o_ref[0, 4 * j:4 * j + 4] = (
            oj.reshape(4, D, A) + rbb[4 * j:4 * j + 4] + entb
        )


def _tc_fused(xT, W_bd, rowbias3, entT, Tt):
    B, T, F, A = xT.shape
    D = rowbias3.shape[-1]
    nT = T // Tt
    return pl.pallas_call(
        _tc_body,
        grid=(B, nT),
        in_specs=[
            pl.BlockSpec((1, Tt, F, A), lambda b, j: (b, j, 0, 0)),
            pl.BlockSpec((4 * D, 4 * F), lambda b, j: (0, 0)),
            pl.BlockSpec((1, Tt, D), lambda b, j: (b, j, 0)),
            pl.BlockSpec((D, A), lambda b, j: (0, 0)),
        ],
        out_specs=pl.BlockSpec((1, Tt, D, A), lambda b, j: (b, j, 0, 0)),
        out_shape=jax.ShapeDtypeStruct((B, T, D, A), jnp.float32),
    )(xT, W_bd, rowbias3, entT)


def kernel(x, x_temp, W, b, t_month, t_weekday, t_hour, t_year, ent_table):
    B, T, A, F = x.shape
    D = W.shape[1]
    BT = B * T
    xt = x_temp.astype(jnp.int32).reshape(BT * 4)
    pe_b = _pe_table(_MAX_LEN, D)[:T] + b[None, :]

    sc_rowbias = _make_sc_rowbias(
        BT, T, D,
        t_month.shape[0], t_weekday.shape[0], t_hour.shape[0], t_year.shape[0],
    )
    rowbias = sc_rowbias(
        xt, t_month.reshape(-1), t_weekday.reshape(-1),
        t_hour.reshape(-1), t_year.reshape(-1), pe_b.reshape(-1),
    ).reshape(B, T, D)

    xT = jnp.transpose(x, (0, 1, 3, 2))
    W_bd = jnp.kron(jnp.eye(4, dtype=jnp.float32), W.T)
    entT = ent_table.T
    outT = _tc_fused(xT, W_bd, rowbias, entT, Tt=32)
    return jnp.transpose(outT, (0, 1, 3, 2))


# speedup vs baseline: 2.1966x; 1.0325x over previous
import functools
import math

import jax
import jax.numpy as jnp
from jax import lax
from jax.experimental import pallas as pl
from jax.experimental.pallas import tpu as pltpu
from jax.experimental.pallas import tpu_sc as plsc


def _make_sc_tem(BT, T, D, n_month, n_weekday, n_hour, n_year):
    mesh = plsc.VectorSubcoreMesh(core_axis_name="c", subcore_axis_name="s")
    NC = mesh.num_cores
    NS = mesh.num_subcores
    NW = NC * NS
    assert BT % NW == 0
    rows_per_w = BT // NW
    assert rows_per_w % 16 == 0 and T % rows_per_w == 0
    L = 16

    @functools.partial(
        pl.kernel,
        out_type=jax.ShapeDtypeStruct((BT * D,), jnp.float32),
        mesh=mesh,
        compiler_params=pltpu.CompilerParams(needs_layout_passes=False),
        scratch_types=[
            pltpu.VMEM((rows_per_w * 4,), jnp.int32),
            pltpu.VMEM((n_month * D,), jnp.float32),
            pltpu.VMEM((n_weekday * D,), jnp.float32),
            pltpu.VMEM((n_hour * D,), jnp.float32),
            pltpu.VMEM((n_year * D,), jnp.float32),
            pltpu.VMEM((rows_per_w * D,), jnp.float32),
        ],
    )
    def sc_tem(
        xt_hbm, mo_hbm, wd_hbm, hr_hbm, yr_hbm, out_hbm,
        idx_v, mo_v, wd_v, hr_v, yr_v, out_v,
    ):
        wid = lax.axis_index("s") * NC + lax.axis_index("c")
        base = wid * rows_per_w
        b_i = base // T
        t0 = jnp.remainder(base, T)
        for k in range(4):
            pltpu.sync_copy(
                xt_hbm.at[pl.ds(b_i * 4 * T + k * T + t0, rows_per_w)],
                idx_v.at[pl.ds(k * rows_per_w, rows_per_w)],
            )
        pltpu.sync_copy(mo_hbm, mo_v)
        pltpu.sync_copy(wd_hbm, wd_v)
        pltpu.sync_copy(hr_hbm, hr_v)
        pltpu.sync_copy(yr_hbm, yr_v)

        lanes = lax.iota(jnp.int32, L)

        def gbody(g, _):
            rows_loc = lanes + g * L
            m_off = plsc.load_gather(idx_v, [rows_loc]) * D
            w_off = plsc.load_gather(idx_v, [rows_loc + rows_per_w]) * D
            h_off = plsc.load_gather(idx_v, [rows_loc + 2 * rows_per_w]) * D
            y_idx = jnp.minimum(
                plsc.load_gather(idx_v, [rows_loc + 3 * rows_per_w]),
                n_year - 1,
            )
            y_off = y_idx * D
            r_off = rows_loc * D
            for d in range(D):
                acc = plsc.load_gather(mo_v, [m_off + d])
                acc = acc + plsc.load_gather(wd_v, [w_off + d])
                acc = acc + plsc.load_gather(hr_v, [h_off + d])
                acc = acc + plsc.load_gather(yr_v, [y_off + d])
                plsc.store_scatter(out_v, [r_off + d], acc)
            return 0

        lax.fori_loop(0, rows_per_w // L, gbody, 0)

        pltpu.sync_copy(out_v, out_hbm.at[pl.ds(base * D, rows_per_w * D)])

    return sc_tem


def _tc_body(x_ref, w_ref, rb_ref, ent_ref, o_ref):
    _, Tt, F, A = x_ref.shape
    D = o_ref.shape[2]
    xm = x_ref[0].reshape(Tt * F, A)
    rbb = lax.broadcast_in_dim(rb_ref[0], (Tt, D, A), (0, 1))
    entb = ent_ref[...][None]
    G = 4 * F
    for j in range(Tt // 4):
        oj = jnp.dot(
            w_ref[...], xm[j * G:(j + 1) * G],
            preferred_element_type=jnp.float32,
        )
        o_ref[0, 4 * j:4 * j + 4] = (
            oj.reshape(4, D, A) + rbb[4 * j:4 * j + 4] + entb
        )


def _tc_fused(xT, W_bd, rowbias3, entT, Tt):
    B, T, F, A = xT.shape
    D = rowbias3.shape[-1]
    nT = T // Tt
    return pl.pallas_call(
        _tc_body,
        grid=(B, nT),
        in_specs=[
            pl.BlockSpec((1, Tt, F, A), lambda b, j: (b, j, 0, 0)),
            pl.BlockSpec((4 * D, 4 * F), lambda b, j: (0, 0)),
            pl.BlockSpec((1, Tt, D), lambda b, j: (b, j, 0)),
            pl.BlockSpec((D, A), lambda b, j: (0, 0)),
        ],
        out_specs=pl.BlockSpec((1, Tt, D, A), lambda b, j: (b, j, 0, 0)),
        out_shape=jax.ShapeDtypeStruct((B, T, D, A), jnp.float32),
    )(xT, W_bd, rowbias3, entT)


def kernel(x, x_temp, W, b, t_month, t_weekday, t_hour, t_year, ent_table):
    B, T, A, F = x.shape
    D = W.shape[1]
    BT = B * T

    xtn = jnp.transpose(x_temp.astype(jnp.int32), (0, 2, 1)).reshape(-1)

    sc_tem = _make_sc_tem(
        BT, T, D,
        t_month.shape[0], t_weekday.shape[0], t_hour.shape[0], t_year.shape[0],
    )
    tem_flat = sc_tem(
        xtn, t_month.reshape(-1), t_weekday.reshape(-1),
        t_hour.reshape(-1), t_year.reshape(-1),
    )

    pos = jnp.arange(T, dtype=jnp.float32)[:, None]
    div = jnp.exp(
        jnp.arange(0, D, 2, dtype=jnp.float32) * (-math.log(10000.0) / D)
    )
    ang = pos * div[None, :]
    pe = jnp.stack([jnp.sin(ang), jnp.cos(ang)], axis=-1).reshape(T, D)
    pe_b = pe + b[None, :]
    rb3 = tem_flat.reshape(B, T, D) + pe_b[None]

    xT = jnp.transpose(x, (0, 1, 3, 2))
    W_bd = jnp.kron(jnp.eye(4, dtype=jnp.float32), W.T)
    entT = ent_table.T
    outT = _tc_fused(xT, W_bd, rb3, entT, Tt=32)
    return jnp.transpose(outT, (0, 1, 3, 2))


# speedup vs baseline: 3.1264x; 1.4233x over previous
import functools
import math

import jax
import jax.numpy as jnp
from jax import lax
from jax.experimental import pallas as pl
from jax.experimental.pallas import tpu as pltpu
from jax.experimental.pallas import tpu_sc as plsc

_DP = 65


def _make_sc_tem(BT, T, D, n_month, n_weekday, n_hour, n_year):
    mesh = plsc.VectorSubcoreMesh(core_axis_name="c", subcore_axis_name="s")
    NC = mesh.num_cores
    NS = mesh.num_subcores
    NW = NC * NS
    assert BT % NW == 0
    rows_per_w = BT // NW
    assert rows_per_w % 16 == 0 and T % rows_per_w == 0
    L = 16

    @functools.partial(
        pl.kernel,
        out_type=jax.ShapeDtypeStruct((BT * _DP,), jnp.float32),
        mesh=mesh,
        compiler_params=pltpu.CompilerParams(needs_layout_passes=False),
        scratch_types=[
            pltpu.VMEM((rows_per_w * 4,), jnp.int32),
            pltpu.VMEM((n_month * _DP,), jnp.float32),
            pltpu.VMEM((n_weekday * _DP,), jnp.float32),
            pltpu.VMEM((n_hour * _DP,), jnp.float32),
            pltpu.VMEM((n_year * _DP,), jnp.float32),
            pltpu.VMEM((rows_per_w * _DP,), jnp.float32),
            pltpu.SemaphoreType.DMA,
        ],
    )
    def sc_tem(
        xt_hbm, mo_hbm, wd_hbm, hr_hbm, yr_hbm, out_hbm,
        idx_v, mo_v, wd_v, hr_v, yr_v, out_v, sem,
    ):
        wid = lax.axis_index("s") * NC + lax.axis_index("c")
        base = wid * rows_per_w
        b_i = base // T
        t0 = jnp.remainder(base, T)
        copies = [
            pltpu.make_async_copy(
                xt_hbm.at[pl.ds(b_i * 4 * T + k * T + t0, rows_per_w)],
                idx_v.at[pl.ds(k * rows_per_w, rows_per_w)],
                sem,
            )
            for k in range(4)
        ] + [
            pltpu.make_async_copy(src, dst, sem)
            for src, dst in
            ((mo_hbm, mo_v), (wd_hbm, wd_v), (hr_hbm, hr_v), (yr_hbm, yr_v))
        ]
        for c in copies:
            c.start()
        for c in copies:
            c.wait()

        lanes = lax.iota(jnp.int32, L)

        def gbody(g, _):
            rows_loc = lanes + g * L
            m_off = plsc.load_gather(idx_v, [rows_loc]) * _DP
            w_off = plsc.load_gather(idx_v, [rows_loc + rows_per_w]) * _DP
            h_off = plsc.load_gather(idx_v, [rows_loc + 2 * rows_per_w]) * _DP
            y_idx = jnp.minimum(
                plsc.load_gather(idx_v, [rows_loc + 3 * rows_per_w]),
                n_year - 1,
            )
            y_off = y_idx * _DP
            r_off = rows_loc * _DP
            for d in range(D):
                acc = plsc.load_gather(mo_v, [m_off + d])
                acc = acc + plsc.load_gather(wd_v, [w_off + d])
                acc = acc + plsc.load_gather(hr_v, [h_off + d])
                acc = acc + plsc.load_gather(yr_v, [y_off + d])
                plsc.store_scatter(out_v, [r_off + d], acc)
            return 0

        lax.fori_loop(0, rows_per_w // L, gbody, 0)

        pltpu.sync_copy(out_v, out_hbm.at[pl.ds(base * _DP, rows_per_w * _DP)])

    return sc_tem


def _tc_body(x_ref, w_ref, rb_ref, ent_ref, o_ref):
    _, Tt, F, A = x_ref.shape
    D = o_ref.shape[2]
    xm = x_ref[0].reshape(Tt * F, A)
    rbb = lax.broadcast_in_dim(rb_ref[0], (Tt, D, A), (0, 1))
    entb = ent_ref[...][None]
    G = 4 * F
    for j in range(Tt // 4):
        oj = jnp.dot(
            w_ref[...], xm[j * G:(j + 1) * G],
            preferred_element_type=jnp.float32,
        )
        o_ref[0, 4 * j:4 * j + 4] = (
            oj.reshape(4, D, A) + rbb[4 * j:4 * j + 4] + entb
        )


def _tc_fused(xT, W_bd, rowbias3, entT, Tt):
    B, T, F, A = xT.shape
    D = rowbias3.shape[-1]
    nT = T // Tt
    return pl.pallas_call(
        _tc_body,
        grid=(B, nT),
        in_specs=[
            pl.BlockSpec((1, Tt, F, A), lambda b, j: (b, j, 0, 0)),
            pl.BlockSpec((4 * D, 4 * F), lambda b, j: (0, 0)),
            pl.BlockSpec((1, Tt, D), lambda b, j: (b, j, 0)),
            pl.BlockSpec((D, A), lambda b, j: (0, 0)),
        ],
        out_specs=pl.BlockSpec((1, Tt, D, A), lambda b, j: (b, j, 0, 0)),
        out_shape=jax.ShapeDtypeStruct((B, T, D, A), jnp.float32),
    )(xT, W_bd, rowbias3, entT)


def kernel(x, x_temp, W, b, t_month, t_weekday, t_hour, t_year, ent_table):
    B, T, A, F = x.shape
    D = W.shape[1]
    BT = B * T

    xtn = jnp.transpose(x_temp.astype(jnp.int32), (0, 2, 1)).reshape(-1)

    def padded(tb):
        return jnp.pad(tb, ((0, 0), (0, _DP - D))).reshape(-1)

    sc_tem = _make_sc_tem(
        BT, T, D,
        t_month.shape[0], t_weekday.shape[0], t_hour.shape[0], t_year.shape[0],
    )
    tem_flat = sc_tem(
        xtn, padded(t_month), padded(t_weekday), padded(t_hour),
        padded(t_year),
    )

    pos = jnp.arange(T, dtype=jnp.float32)[:, None]
    div = jnp.exp(
        jnp.arange(0, D, 2, dtype=jnp.float32) * (-math.log(10000.0) / D)
    )
    ang = pos * div[None, :]
    pe = jnp.stack([jnp.sin(ang), jnp.cos(ang)], axis=-1).reshape(T, D)
    pe_b = pe + b[None, :]
    rb3 = tem_flat.reshape(B, T, _DP)[:, :, :D] + pe_b[None]

    xT = jnp.transpose(x, (0, 1, 3, 2))
    W_bd = jnp.kron(jnp.eye(4, dtype=jnp.float32), W.T)
    entT = ent_table.T
    outT = _tc_fused(xT, W_bd, rb3, entT, Tt=64)
    return jnp.transpose(outT, (0, 1, 3, 2))


# speedup vs baseline: 3.5568x; 1.1377x over previous
import functools
import math

import jax
import jax.numpy as jnp
from jax import lax
from jax.experimental import pallas as pl
from jax.experimental.pallas import tpu as pltpu
from jax.experimental.pallas import tpu_sc as plsc

_DP = 65


def _make_sc_tem(BT, T, D, n_month, n_weekday, n_hour, n_year):
    mesh = plsc.VectorSubcoreMesh(core_axis_name="c", subcore_axis_name="s")
    NC = mesh.num_cores
    NS = mesh.num_subcores
    NW = NC * NS
    assert BT % NW == 0
    rows_per_w = BT // NW
    assert rows_per_w % 16 == 0 and T % rows_per_w == 0
    L = 16

    @functools.partial(
        pl.kernel,
        out_type=jax.ShapeDtypeStruct((BT * _DP,), jnp.float32),
        mesh=mesh,
        compiler_params=pltpu.CompilerParams(needs_layout_passes=False),
        scratch_types=[
            pltpu.VMEM((rows_per_w * 4,), jnp.int32),
            pltpu.VMEM((n_month * _DP,), jnp.float32),
            pltpu.VMEM((n_weekday * _DP,), jnp.float32),
            pltpu.VMEM((n_hour * _DP,), jnp.float32),
            pltpu.VMEM((n_year * _DP,), jnp.float32),
            pltpu.VMEM((rows_per_w * _DP,), jnp.float32),
            pltpu.SemaphoreType.DMA,
        ],
    )
    def sc_tem(
        xt_hbm, mo_hbm, wd_hbm, hr_hbm, yr_hbm, out_hbm,
        idx_v, mo_v, wd_v, hr_v, yr_v, out_v, sem,
    ):
        wid = lax.axis_index("s") * NC + lax.axis_index("c")
        base = wid * rows_per_w
        b_i = base // T
        t0 = jnp.remainder(base, T)
        copies = [
            pltpu.make_async_copy(
                xt_hbm.at[pl.ds(b_i * 4 * T + k * T + t0, rows_per_w)],
                idx_v.at[pl.ds(k * rows_per_w, rows_per_w)],
                sem,
            )
            for k in range(4)
        ] + [
            pltpu.make_async_copy(src, dst, sem)
            for src, dst in
            ((mo_hbm, mo_v), (wd_hbm, wd_v), (hr_hbm, hr_v), (yr_hbm, yr_v))
        ]
        for c in copies:
            c.start()
        for c in copies:
            c.wait()

        lanes = lax.iota(jnp.int32, L)

        def gbody(g, _):
            rows_loc = lanes + g * L
            m_off = plsc.load_gather(idx_v, [rows_loc]) * _DP
            w_off = plsc.load_gather(idx_v, [rows_loc + rows_per_w]) * _DP
            h_off = plsc.load_gather(idx_v, [rows_loc + 2 * rows_per_w]) * _DP
            y_idx = jnp.minimum(
                plsc.load_gather(idx_v, [rows_loc + 3 * rows_per_w]),
                n_year - 1,
            )
            y_off = y_idx * _DP
            r_off = rows_loc * _DP
            for d in range(D):
                acc = plsc.load_gather(mo_v, [m_off + d])
                acc = acc + plsc.load_gather(wd_v, [w_off + d])
                acc = acc + plsc.load_gather(hr_v, [h_off + d])
                acc = acc + plsc.load_gather(yr_v, [y_off + d])
                plsc.store_scatter(out_v, [r_off + d], acc)
            return 0

        lax.fori_loop(0, rows_per_w // L, gbody, 0)

        pltpu.sync_copy(out_v, out_hbm.at[pl.ds(base * _DP, rows_per_w * _DP)])

    return sc_tem


def _tc_body(x_ref, w_ref, rb_ref, ent_ref, o_ref):
    _, Tt, F, A = x_ref.shape
    D = o_ref.shape[2]
    xm = x_ref[0].reshape(Tt * F, A)
    rbb = lax.broadcast_in_dim(rb_ref[0], (Tt, D, A), (0, 1))
    entb = ent_ref[...][None]
    G = 4 * F
    for j in range(Tt // 4):
        oj = jnp.dot(
            w_ref[...], xm[j * G:(j + 1) * G],
            preferred_element_type=jnp.float32,
        )
        o_ref[0, 4 * j:4 * j + 4] = (
            oj.reshape(4, D, A) + rbb[4 * j:4 * j + 4] + entb
        )


def _tc_fused(xT, W_bd, rowbias3, entT, Tt):
    B, T, F, A = xT.shape
    D = rowbias3.shape[-1]
    nT = T // Tt
    return pl.pallas_call(
        _tc_body,
        grid=(B, nT),
        in_specs=[
            pl.BlockSpec((1, Tt, F, A), lambda b, j: (b, j, 0, 0)),
            pl.BlockSpec((4 * D, 4 * F), lambda b, j: (0, 0)),
            pl.BlockSpec((1, Tt, D), lambda b, j: (b, j, 0)),
            pl.BlockSpec((D, A), lambda b, j: (0, 0)),
        ],
        out_specs=pl.BlockSpec((1, Tt, D, A), lambda b, j: (b, j, 0, 0)),
        out_shape=jax.ShapeDtypeStruct((B, T, D, A), jnp.float32),
    )(xT, W_bd, rowbias3, entT)


def kernel(x, x_temp, W, b, t_month, t_weekday, t_hour, t_year, ent_table):
    B, T, A, F = x.shape
    D = W.shape[1]
    BT = B * T

    xtn = jnp.transpose(x_temp.astype(jnp.int32), (0, 2, 1)).reshape(-1)

    def padded(tb):
        return jnp.pad(tb, ((0, 0), (0, _DP - D))).reshape(-1)

    sc_tem = _make_sc_tem(
        BT, T, D,
        t_month.shape[0], t_weekday.shape[0], t_hour.shape[0], t_year.shape[0],
    )
    tem_flat = sc_tem(
        xtn, padded(t_month), padded(t_weekday), padded(t_hour),
        padded(t_year),
    )

    pos = jnp.arange(T, dtype=jnp.float32)[:, None]
    div = jnp.exp(
        jnp.arange(0, D, 2, dtype=jnp.float32) * (-math.log(10000.0) / D)
    )
    ang = pos * div[None, :]
    pe = jnp.stack([jnp.sin(ang), jnp.cos(ang)], axis=-1).reshape(T, D)
    pe_b = pe + b[None, :]
    rb3 = tem_flat.reshape(B, T, _DP)[:, :, :D] + pe_b[None]

    xT = jnp.transpose(x, (0, 1, 3, 2))
    W_bd = jnp.kron(jnp.eye(4, dtype=jnp.float32), W.T)
    entT = ent_table.T
    outT = _tc_fused(xT, W_bd, rb3, entT, Tt=128)
    return jnp.transpose(outT, (0, 1, 3, 2))


# speedup vs baseline: 3.5907x; 1.0095x over previous
import functools
import math

import jax
import jax.numpy as jnp
from jax import lax
from jax.experimental import pallas as pl
from jax.experimental.pallas import tpu as pltpu
from jax.experimental.pallas import tpu_sc as plsc

_DP = 65


def _make_sc_tem(BT, T, D, n_month, n_weekday, n_hour, n_year):
    mesh = plsc.VectorSubcoreMesh(core_axis_name="c", subcore_axis_name="s")
    NC = mesh.num_cores
    NS = mesh.num_subcores
    NW = NC * NS
    assert BT % NW == 0
    rows_per_w = BT // NW
    assert rows_per_w % 16 == 0 and T % rows_per_w == 0
    L = 16

    @functools.partial(
        pl.kernel,
        out_type=jax.ShapeDtypeStruct((BT * _DP,), jnp.float32),
        mesh=mesh,
        compiler_params=pltpu.CompilerParams(needs_layout_passes=False),
        scratch_types=[
            pltpu.VMEM((rows_per_w * 4,), jnp.int32),
            pltpu.VMEM((n_month * _DP,), jnp.float32),
            pltpu.VMEM((n_weekday * _DP,), jnp.float32),
            pltpu.VMEM((n_hour * _DP,), jnp.float32),
            pltpu.VMEM((n_year * _DP,), jnp.float32),
            pltpu.VMEM((rows_per_w * _DP,), jnp.float32),
            pltpu.SemaphoreType.DMA,
        ],
    )
    def sc_tem(
        xt_hbm, mo_hbm, wd_hbm, hr_hbm, yr_hbm, out_hbm,
        idx_v, mo_v, wd_v, hr_v, yr_v, out_v, sem,
    ):
        wid = lax.axis_index("s") * NC + lax.axis_index("c")
        base = wid * rows_per_w
        b_i = base // T
        t0 = jnp.remainder(base, T)
        copies = [
            pltpu.make_async_copy(
                xt_hbm.at[pl.ds(b_i * 4 * T + k * T + t0, rows_per_w)],
                idx_v.at[pl.ds(k * rows_per_w, rows_per_w)],
                sem,
            )
            for k in range(4)
        ] + [
            pltpu.make_async_copy(src, dst, sem)
            for src, dst in
            ((mo_hbm, mo_v), (wd_hbm, wd_v), (hr_hbm, hr_v), (yr_hbm, yr_v))
        ]
        for c in copies:
            c.start()
        for c in copies:
            c.wait()

        lanes = lax.iota(jnp.int32, L)

        def gbody(g, _):
            rows_loc = lanes + g * L
            m_off = plsc.load_gather(idx_v, [rows_loc]) * _DP
            w_off = plsc.load_gather(idx_v, [rows_loc + rows_per_w]) * _DP
            h_off = plsc.load_gather(idx_v, [rows_loc + 2 * rows_per_w]) * _DP
            y_idx = jnp.minimum(
                plsc.load_gather(idx_v, [rows_loc + 3 * rows_per_w]),
                n_year - 1,
            )
            y_off = y_idx * _DP
            r_off = rows_loc * _DP
            for d in range(D):
                acc = plsc.load_gather(mo_v, [m_off + d])
                acc = acc + plsc.load_gather(wd_v, [w_off + d])
                acc = acc + plsc.load_gather(hr_v, [h_off + d])
                acc = acc + plsc.load_gather(yr_v, [y_off + d])
                plsc.store_scatter(out_v, [r_off + d], acc)
            return 0

        lax.fori_loop(0, rows_per_w // L, gbody, 0)

        pltpu.sync_copy(out_v, out_hbm.at[pl.ds(base * _DP, rows_per_w * _DP)])

    return sc_tem


def _tc_body(x_ref, w_ref, rb_ref, ent_ref, o_ref):
    _, Tt, F, A = x_ref.shape
    D = o_ref.shape[2]
    xm = x_ref[0].reshape(Tt * F, A)
    rbb = lax.broadcast_in_dim(rb_ref[0], (Tt, D, A), (0, 1))
    entb = ent_ref[...][None]
    G = 4 * F
    for j in range(Tt // 4):
        oj = jnp.dot(
            w_ref[...], xm[j * G:(j + 1) * G],
            preferred_element_type=jnp.float32,
        )
        o_ref[0, 4 * j:4 * j + 4] = (
            oj.reshape(4, D, A) + rbb[4 * j:4 * j + 4] + entb
        )


def _tc_fused(xT, W_bd, rowbias3, entT, Tt):
    B, T, F, A = xT.shape
    D = rowbias3.shape[-1]
    nT = T // Tt
    return pl.pallas_call(
        _tc_body,
        grid=(B, nT),
        in_specs=[
            pl.BlockSpec((1, Tt, F, A), lambda b, j: (b, j, 0, 0)),
            pl.BlockSpec((4 * D, 4 * F), lambda b, j: (0, 0)),
            pl.BlockSpec((1, Tt, D), lambda b, j: (b, j, 0)),
            pl.BlockSpec((D, A), lambda b, j: (0, 0)),
        ],
        out_specs=pl.BlockSpec((1, Tt, D, A), lambda b, j: (b, j, 0, 0)),
        out_shape=jax.ShapeDtypeStruct((B, T, D, A), jnp.float32),
    )(xT, W_bd, rowbias3, entT)


def kernel(x, x_temp, W, b, t_month, t_weekday, t_hour, t_year, ent_table):
    B, T, A, F = x.shape
    D = W.shape[1]
    BT = B * T

    xtn = jnp.transpose(x_temp.astype(jnp.int32), (0, 2, 1)).reshape(-1)

    def padded(tb):
        return jnp.pad(tb, ((0, 0), (0, _DP - D))).reshape(-1)

    sc_tem = _make_sc_tem(
        BT, T, D,
        t_month.shape[0], t_weekday.shape[0], t_hour.shape[0], t_year.shape[0],
    )
    tem_flat = sc_tem(
        xtn, padded(t_month), padded(t_weekday), padded(t_hour),
        padded(t_year),
    )

    pos = jnp.arange(T, dtype=jnp.float32)[:, None]
    div = jnp.exp(
        jnp.arange(0, D, 2, dtype=jnp.float32) * (-math.log(10000.0) / D)
    )
    ang = pos * div[None, :]
    pe = jnp.stack([jnp.sin(ang), jnp.cos(ang)], axis=-1).reshape(T, D)
    pe_b = pe + b[None, :]
    rb3 = tem_flat.reshape(B, T, _DP)[:, :, :D] + pe_b[None]

    xT = jnp.transpose(x, (0, 1, 3, 2))
    W_bd = jnp.kron(jnp.eye(4, dtype=jnp.float32), W.T)
    entT = ent_table.T
    outT = _tc_fused(xT, W_bd, rb3, entT, Tt=256)
    return jnp.transpose(outT, (0, 1, 3, 2))


# speedup vs baseline: 3.6597x; 1.0192x over previous
import functools
import math

import jax
import jax.numpy as jnp
from jax import lax
from jax.experimental import pallas as pl
from jax.experimental.pallas import tpu as pltpu
from jax.experimental.pallas import tpu_sc as plsc

def _make_sc_tem(BT, T, D, n_month, n_weekday, n_hour, n_year):
    mesh = plsc.VectorSubcoreMesh(core_axis_name="c", subcore_axis_name="s")
    NC = mesh.num_cores
    NS = mesh.num_subcores
    NW = NC * NS
    assert BT % NW == 0
    rows_per_w = BT // NW
    assert rows_per_w % 16 == 0 and T % rows_per_w == 0
    L = 16

    @functools.partial(
        pl.kernel,
        out_type=jax.ShapeDtypeStruct((BT * D,), jnp.float32),
        mesh=mesh,
        compiler_params=pltpu.CompilerParams(needs_layout_passes=False),
        scratch_types=[
            pltpu.VMEM((rows_per_w * 4,), jnp.int32),
            pltpu.VMEM((n_month * D,), jnp.float32),
            pltpu.VMEM((n_weekday * D,), jnp.float32),
            pltpu.VMEM((n_hour * D,), jnp.float32),
            pltpu.VMEM((n_year * D,), jnp.float32),
            pltpu.VMEM((rows_per_w * D,), jnp.float32),
            pltpu.SemaphoreType.DMA,
        ],
    )
    def sc_tem(
        xt_hbm, mo_hbm, wd_hbm, hr_hbm, yr_hbm, out_hbm,
        idx_v, mo_v, wd_v, hr_v, yr_v, out_v, sem,
    ):
        wid = lax.axis_index("s") * NC + lax.axis_index("c")
        base = wid * rows_per_w
        b_i = base // T
        t0 = jnp.remainder(base, T)
        copies = [
            pltpu.make_async_copy(
                xt_hbm.at[pl.ds(b_i * 4 * T + k * T + t0, rows_per_w)],
                idx_v.at[pl.ds(k * rows_per_w, rows_per_w)],
                sem,
            )
            for k in range(4)
        ] + [
            pltpu.make_async_copy(src, dst, sem)
            for src, dst in
            ((mo_hbm, mo_v), (wd_hbm, wd_v), (hr_hbm, hr_v), (yr_hbm, yr_v))
        ]
        for c in copies:
            c.start()
        for c in copies:
            c.wait()

        lanes = lax.iota(jnp.int32, L)

        def gbody(g, _):
            rows_loc = lanes + g * L
            m_off = plsc.load_gather(idx_v, [rows_loc]) * D
            w_off = plsc.load_gather(idx_v, [rows_loc + rows_per_w]) * D
            h_off = plsc.load_gather(idx_v, [rows_loc + 2 * rows_per_w]) * D
            y_idx = jnp.minimum(
                plsc.load_gather(idx_v, [rows_loc + 3 * rows_per_w]),
                n_year - 1,
            )
            y_off = y_idx * D
            r_off = rows_loc * D
            for d in range(D):
                dcol = jnp.bitwise_and(d + lanes, D - 1)
                acc = plsc.load_gather(mo_v, [m_off + dcol])
                acc = acc + plsc.load_gather(wd_v, [w_off + dcol])
                acc = acc + plsc.load_gather(hr_v, [h_off + dcol])
                acc = acc + plsc.load_gather(yr_v, [y_off + dcol])
                plsc.store_scatter(out_v, [r_off + dcol], acc)
            return 0

        lax.fori_loop(0, rows_per_w // L, gbody, 0)

        pltpu.sync_copy(out_v, out_hbm.at[pl.ds(base * D, rows_per_w * D)])

    return sc_tem


def _tc_body(x_ref, w_ref, rb_ref, ent_ref, o_ref):
    _, Tt, F, A = x_ref.shape
    D = o_ref.shape[2]
    xm = x_ref[0].reshape(Tt * F, A)
    rbb = lax.broadcast_in_dim(rb_ref[0], (Tt, D, A), (0, 1))
    entb = ent_ref[...][None]
    G = 4 * F
    for j in range(Tt // 4):
        oj = jnp.dot(
            w_ref[...], xm[j * G:(j + 1) * G],
            preferred_element_type=jnp.float32,
        )
        o_ref[0, 4 * j:4 * j + 4] = (
            oj.reshape(4, D, A) + rbb[4 * j:4 * j + 4] + entb
        )


def _tc_fused(xT, W_bd, rowbias3, entT, Tt):
    B, T, F, A = xT.shape
    D = rowbias3.shape[-1]
    nT = T // Tt
    return pl.pallas_call(
        _tc_body,
        grid=(B, nT),
        in_specs=[
            pl.BlockSpec((1, Tt, F, A), lambda b, j: (b, j, 0, 0)),
            pl.BlockSpec((4 * D, 4 * F), lambda b, j: (0, 0)),
            pl.BlockSpec((1, Tt, D), lambda b, j: (b, j, 0)),
            pl.BlockSpec((D, A), lambda b, j: (0, 0)),
        ],
        out_specs=pl.BlockSpec((1, Tt, D, A), lambda b, j: (b, j, 0, 0)),
        out_shape=jax.ShapeDtypeStruct((B, T, D, A), jnp.float32),
    )(xT, W_bd, rowbias3, entT)


def kernel(x, x_temp, W, b, t_month, t_weekday, t_hour, t_year, ent_table):
    B, T, A, F = x.shape
    D = W.shape[1]
    BT = B * T

    xtn = jnp.transpose(x_temp.astype(jnp.int32), (0, 2, 1)).reshape(-1)

    sc_tem = _make_sc_tem(
        BT, T, D,
        t_month.shape[0], t_weekday.shape[0], t_hour.shape[0], t_year.shape[0],
    )
    tem_flat = sc_tem(
        xtn, t_month.reshape(-1), t_weekday.reshape(-1), t_hour.reshape(-1),
        t_year.reshape(-1),
    )

    pos = jnp.arange(T, dtype=jnp.float32)[:, None]
    div = jnp.exp(
        jnp.arange(0, D, 2, dtype=jnp.float32) * (-math.log(10000.0) / D)
    )
    ang = pos * div[None, :]
    pe = jnp.stack([jnp.sin(ang), jnp.cos(ang)], axis=-1).reshape(T, D)
    pe_b = pe + b[None, :]
    rb3 = tem_flat.reshape(B, T, D) + pe_b[None]

    xT = jnp.transpose(x, (0, 1, 3, 2))
    W_bd = jnp.kron(jnp.eye(4, dtype=jnp.float32), W.T)
    entT = ent_table.T
    outT = _tc_fused(xT, W_bd, rb3, entT, Tt=256)
    return jnp.transpose(outT, (0, 1, 3, 2))


# speedup vs baseline: 3.7175x; 1.0158x over previous
import functools
import math

import jax
import jax.numpy as jnp
from jax import lax
from jax.experimental import pallas as pl
from jax.experimental.pallas import tpu as pltpu
from jax.experimental.pallas import tpu_sc as plsc

def _make_sc_tem(BT, T, D, n_month, n_weekday, n_hour, n_year):
    mesh = plsc.VectorSubcoreMesh(core_axis_name="c", subcore_axis_name="s")
    NC = mesh.num_cores
    NS = mesh.num_subcores
    NW = NC * NS
    assert BT % NW == 0
    rows_per_w = BT // NW
    assert rows_per_w % 16 == 0 and T % rows_per_w == 0
    L = 16

    n_tab = n_month + n_weekday + n_hour + n_year
    wd_base = n_month * D
    hr_base = (n_month + n_weekday) * D
    yr_base = (n_month + n_weekday + n_hour) * D

    @functools.partial(
        pl.kernel,
        out_type=jax.ShapeDtypeStruct((BT * D,), jnp.float32),
        mesh=mesh,
        compiler_params=pltpu.CompilerParams(needs_layout_passes=False),
        scratch_types=[
            pltpu.VMEM((rows_per_w * 4,), jnp.int32),
            pltpu.VMEM((n_tab * D,), jnp.float32),
            pltpu.VMEM((rows_per_w * D,), jnp.float32),
            pltpu.SemaphoreType.DMA,
        ],
    )
    def sc_tem(xt_hbm, tab_hbm, out_hbm, idx_v, tab_v, out_v, sem):
        wid = lax.axis_index("s") * NC + lax.axis_index("c")
        base = wid * rows_per_w
        b_i = base // T
        t0 = jnp.remainder(base, T)
        copies = [
            pltpu.make_async_copy(
                xt_hbm.at[pl.ds(b_i * 4 * T + k * T + t0, rows_per_w)],
                idx_v.at[pl.ds(k * rows_per_w, rows_per_w)],
                sem,
            )
            for k in range(4)
        ] + [pltpu.make_async_copy(tab_hbm, tab_v, sem)]
        for c in copies:
            c.start()
        for c in copies:
            c.wait()

        lanes = lax.iota(jnp.int32, L)

        def gbody(g, _):
            rows_loc = lanes + g * L
            m_off = plsc.load_gather(idx_v, [rows_loc]) * D
            w_off = plsc.load_gather(idx_v, [rows_loc + rows_per_w]) * D + wd_base
            h_off = plsc.load_gather(idx_v, [rows_loc + 2 * rows_per_w]) * D + hr_base
            y_idx = jnp.minimum(
                plsc.load_gather(idx_v, [rows_loc + 3 * rows_per_w]),
                n_year - 1,
            )
            y_off = y_idx * D + yr_base
            r_off = rows_loc * D
            for d in range(D):
                dcol = jnp.bitwise_and(d + lanes, D - 1)
                acc = plsc.load_gather(tab_v, [m_off + dcol])
                acc = acc + plsc.load_gather(tab_v, [w_off + dcol])
                acc = acc + plsc.load_gather(tab_v, [h_off + dcol])
                acc = acc + plsc.load_gather(tab_v, [y_off + dcol])
                plsc.store_scatter(out_v, [r_off + dcol], acc)
            return 0

        lax.fori_loop(0, rows_per_w // L, gbody, 0)

        pltpu.sync_copy(out_v, out_hbm.at[pl.ds(base * D, rows_per_w * D)])

    return sc_tem


def _tc_body(x_ref, w_ref, rb_ref, ent_ref, o_ref):
    _, Tt, F, A = x_ref.shape
    D = o_ref.shape[2]
    xm = x_ref[0].reshape(Tt * F, A)
    rbb = lax.broadcast_in_dim(rb_ref[0], (Tt, D, A), (0, 1))
    entb = ent_ref[...][None]
    G = 4 * F
    for j in range(Tt // 4):
        oj = jnp.dot(
            w_ref[...], xm[j * G:(j + 1) * G],
            preferred_element_type=jnp.float32,
        )
        o_ref[0, 4 * j:4 * j + 4] = (
            oj.reshape(4, D, A) + rbb[4 * j:4 * j + 4] + entb
        )


def _tc_fused(xT, W_bd, rowbias3, entT, Tt):
    B, T, F, A = xT.shape
    D = rowbias3.shape[-1]
    nT = T // Tt
    return pl.pallas_call(
        _tc_body,
        grid=(B, nT),
        in_specs=[
            pl.BlockSpec((1, Tt, F, A), lambda b, j: (b, j, 0, 0)),
            pl.BlockSpec((4 * D, 4 * F), lambda b, j: (0, 0)),
            pl.BlockSpec((1, Tt, D), lambda b, j: (b, j, 0)),
            pl.BlockSpec((D, A), lambda b, j: (0, 0)),
        ],
        out_specs=pl.BlockSpec((1, Tt, D, A), lambda b, j: (b, j, 0, 0)),
        out_shape=jax.ShapeDtypeStruct((B, T, D, A), jnp.float32),
    )(xT, W_bd, rowbias3, entT)


def kernel(x, x_temp, W, b, t_month, t_weekday, t_hour, t_year, ent_table):
    B, T, A, F = x.shape
    D = W.shape[1]
    BT = B * T

    xtn = jnp.transpose(x_temp.astype(jnp.int32), (0, 2, 1)).reshape(-1)

    sc_tem = _make_sc_tem(
        BT, T, D,
        t_month.shape[0], t_weekday.shape[0], t_hour.shape[0], t_year.shape[0],
    )
    tab_cat = jnp.concatenate(
        [t_month.reshape(-1), t_weekday.reshape(-1), t_hour.reshape(-1),
         t_year.reshape(-1)]
    )
    tem_flat = sc_tem(xtn, tab_cat)

    pos = jnp.arange(T, dtype=jnp.float32)[:, None]
    div = jnp.exp(
        jnp.arange(0, D, 2, dtype=jnp.float32) * (-math.log(10000.0) / D)
    )
    ang = pos * div[None, :]
    pe = jnp.stack([jnp.sin(ang), jnp.cos(ang)], axis=-1).reshape(T, D)
    pe_b = pe + b[None, :]
    rb3 = tem_flat.reshape(B, T, D) + pe_b[None]

    xT = jnp.transpose(x, (0, 1, 3, 2))
    W_bd = jnp.kron(jnp.eye(4, dtype=jnp.float32), W.T)
    entT = ent_table.T
    outT = _tc_fused(xT, W_bd, rb3, entT, Tt=256)
    return jnp.transpose(outT, (0, 1, 3, 2))


# speedup vs baseline: 3.7624x; 1.0121x over previous
import functools
import math

import jax
import jax.numpy as jnp
from jax import lax
from jax.experimental import pallas as pl
from jax.experimental.pallas import tpu as pltpu
from jax.experimental.pallas import tpu_sc as plsc

def _make_sc_tem(BT, T, D, n_month, n_weekday, n_hour, n_year):
    mesh = plsc.VectorSubcoreMesh(core_axis_name="c", subcore_axis_name="s")
    NC = mesh.num_cores
    NS = mesh.num_subcores
    NW = NC * NS
    assert BT % NW == 0
    rows_per_w = BT // NW
    assert rows_per_w % 16 == 0 and T % rows_per_w == 0
    L = 16

    n_tab = n_month + n_weekday + n_hour + n_year
    wd_base = n_month * D
    hr_base = (n_month + n_weekday) * D
    yr_base = (n_month + n_weekday + n_hour) * D

    @functools.partial(
        pl.kernel,
        out_type=jax.ShapeDtypeStruct((BT * D,), jnp.float32),
        mesh=mesh,
        compiler_params=pltpu.CompilerParams(needs_layout_passes=False),
        scratch_types=[
            pltpu.VMEM((rows_per_w * 4,), jnp.int32),
            pltpu.VMEM((n_tab * D,), jnp.float32),
            pltpu.VMEM((rows_per_w * D,), jnp.float32),
            pltpu.SemaphoreType.DMA,
        ],
    )
    def sc_tem(xt_hbm, tab_hbm, out_hbm, idx_v, tab_v, out_v, sem):
        wid = lax.axis_index("s") * NC + lax.axis_index("c")
        base = wid * rows_per_w
        b_i = base // T
        t0 = jnp.remainder(base, T)
        copies = [
            pltpu.make_async_copy(
                xt_hbm.at[pl.ds(b_i * 4 * T + k * T + t0, rows_per_w)],
                idx_v.at[pl.ds(k * rows_per_w, rows_per_w)],
                sem,
            )
            for k in range(4)
        ] + [pltpu.make_async_copy(tab_hbm, tab_v, sem)]
        for c in copies:
            c.start()
        for c in copies:
            c.wait()

        lanes = lax.iota(jnp.int32, L)

        def gbody(g, _):
            rows_loc = lanes + g * L
            m_off = plsc.load_gather(idx_v, [rows_loc]) * D
            w_off = plsc.load_gather(idx_v, [rows_loc + rows_per_w]) * D + wd_base
            h_off = plsc.load_gather(idx_v, [rows_loc + 2 * rows_per_w]) * D + hr_base
            y_idx = jnp.minimum(
                plsc.load_gather(idx_v, [rows_loc + 3 * rows_per_w]),
                n_year - 1,
            )
            y_off = y_idx * D + yr_base
            r_off = rows_loc * D

            def dbody(dc, _):
                for u in range(8):
                    dcol = jnp.bitwise_and(dc * 8 + u + lanes, D - 1)
                    acc = plsc.load_gather(tab_v, [m_off + dcol])
                    acc = acc + plsc.load_gather(tab_v, [w_off + dcol])
                    acc = acc + plsc.load_gather(tab_v, [h_off + dcol])
                    acc = acc + plsc.load_gather(tab_v, [y_off + dcol])
                    plsc.store_scatter(out_v, [r_off + dcol], acc)
                return 0

            lax.fori_loop(0, D // 8, dbody, 0)
            return 0

        lax.fori_loop(0, rows_per_w // L, gbody, 0)

        pltpu.sync_copy(out_v, out_hbm.at[pl.ds(base * D, rows_per_w * D)])

    return sc_tem


def _tc_body(x_ref, w_ref, rb_ref, ent_ref, o_ref):
    _, Tt, F, A = x_ref.shape
    D = o_ref.shape[2]
    xm = x_ref[0].reshape(Tt * F, A)
    rbb = lax.broadcast_in_dim(rb_ref[0], (Tt, D, A), (0, 1))
    entb = ent_ref[...][None]
    G = 4 * F
    for j in range(Tt // 4):
        oj = jnp.dot(
            w_ref[...], xm[j * G:(j + 1) * G],
            preferred_element_type=jnp.float32,
        )
        o_ref[0, 4 * j:4 * j + 4] = (
            oj.reshape(4, D, A) + rbb[4 * j:4 * j + 4] + entb
        )


def _tc_fused(xT, W_bd, rowbias3, entT, Tt):
    B, T, F, A = xT.shape
    D = rowbias3.shape[-1]
    nT = T // Tt
    return pl.pallas_call(
        _tc_body,
        grid=(B, nT),
        in_specs=[
            pl.BlockSpec((1, Tt, F, A), lambda b, j: (b, j, 0, 0)),
            pl.BlockSpec((4 * D, 4 * F), lambda b, j: (0, 0)),
            pl.BlockSpec((1, Tt, D), lambda b, j: (b, j, 0)),
            pl.BlockSpec((D, A), lambda b, j: (0, 0)),
        ],
        out_specs=pl.BlockSpec((1, Tt, D, A), lambda b, j: (b, j, 0, 0)),
        out_shape=jax.ShapeDtypeStruct((B, T, D, A), jnp.float32),
    )(xT, W_bd, rowbias3, entT)


def kernel(x, x_temp, W, b, t_month, t_weekday, t_hour, t_year, ent_table):
    B, T, A, F = x.shape
    D = W.shape[1]
    BT = B * T

    xtn = jnp.transpose(x_temp.astype(jnp.int32), (0, 2, 1)).reshape(-1)

    sc_tem = _make_sc_tem(
        BT, T, D,
        t_month.shape[0], t_weekday.shape[0], t_hour.shape[0], t_year.shape[0],
    )
    tab_cat = jnp.concatenate(
        [t_month.reshape(-1), t_weekday.reshape(-1), t_hour.reshape(-1),
         t_year.reshape(-1)]
    )
    tem_flat = sc_tem(xtn, tab_cat)

    pos = jnp.arange(T, dtype=jnp.float32)[:, None]
    div = jnp.exp(
        jnp.arange(0, D, 2, dtype=jnp.float32) * (-math.log(10000.0) / D)
    )
    ang = pos * div[None, :]
    pe = jnp.stack([jnp.sin(ang), jnp.cos(ang)], axis=-1).reshape(T, D)
    pe_b = pe + b[None, :]
    rb3 = tem_flat.reshape(B, T, D) + pe_b[None]

    xT = jnp.transpose(x, (0, 1, 3, 2))
    W_bd = jnp.kron(jnp.eye(4, dtype=jnp.float32), W.T)
    entT = ent_table.T
    outT = _tc_fused(xT, W_bd, rb3, entT, Tt=256)
    return jnp.transpose(outT, (0, 1, 3, 2))


# speedup vs baseline: 3.8202x; 1.0154x over previous
import functools
import math

import jax
import jax.numpy as jnp
from jax import lax
from jax.experimental import pallas as pl
from jax.experimental.pallas import tpu as pltpu
from jax.experimental.pallas import tpu_sc as plsc

def _make_sc_tem(BT, T, D, n_month, n_weekday, n_hour, n_year):
    mesh = plsc.VectorSubcoreMesh(core_axis_name="c", subcore_axis_name="s")
    NC = mesh.num_cores
    NS = mesh.num_subcores
    NW = NC * NS
    assert BT % NW == 0
    rows_per_w = BT // NW
    assert rows_per_w % 16 == 0 and T % rows_per_w == 0
    L = 16

    n_tab = n_month + n_weekday + n_hour + n_year
    wd_base = n_month * D
    hr_base = (n_month + n_weekday) * D
    yr_base = (n_month + n_weekday + n_hour) * D

    @functools.partial(
        pl.kernel,
        out_type=jax.ShapeDtypeStruct((BT * D,), jnp.float32),
        mesh=mesh,
        compiler_params=pltpu.CompilerParams(needs_layout_passes=False),
        scratch_types=[
            pltpu.VMEM((rows_per_w * 4,), jnp.int32),
            pltpu.VMEM((n_tab * D,), jnp.float32),
            pltpu.VMEM((rows_per_w * D,), jnp.float32),
            pltpu.SemaphoreType.DMA,
        ],
    )
    def sc_tem(xt_hbm, tab_hbm, out_hbm, idx_v, tab_v, out_v, sem):
        wid = lax.axis_index("s") * NC + lax.axis_index("c")
        base = wid * rows_per_w
        b_i = base // T
        t0 = jnp.remainder(base, T)
        copies = [
            pltpu.make_async_copy(
                xt_hbm.at[pl.ds(b_i * 4 * T + k * T + t0, rows_per_w)],
                idx_v.at[pl.ds(k * rows_per_w, rows_per_w)],
                sem,
            )
            for k in range(4)
        ] + [pltpu.make_async_copy(tab_hbm, tab_v, sem)]
        for c in copies:
            c.start()
        for c in copies:
            c.wait()

        lanes = lax.iota(jnp.int32, L)

        def gbody(g, _):
            rows_loc = lanes + g * L
            m_off = plsc.load_gather(idx_v, [rows_loc]) * D
            w_off = plsc.load_gather(idx_v, [rows_loc + rows_per_w]) * D + wd_base
            h_off = plsc.load_gather(idx_v, [rows_loc + 2 * rows_per_w]) * D + hr_base
            y_idx = jnp.minimum(
                plsc.load_gather(idx_v, [rows_loc + 3 * rows_per_w]),
                n_year - 1,
            )
            y_off = y_idx * D + yr_base
            r_off = rows_loc * D

            def dbody(dc, _):
                for u in range(8):
                    dcol = jnp.bitwise_and(dc * 8 + u + lanes, D - 1)
                    acc = plsc.load_gather(tab_v, [m_off + dcol])
                    acc = acc + plsc.load_gather(tab_v, [w_off + dcol])
                    acc = acc + plsc.load_gather(tab_v, [h_off + dcol])
                    acc = acc + plsc.load_gather(tab_v, [y_off + dcol])
                    plsc.store_scatter(out_v, [r_off + dcol], acc)
                return 0

            lax.fori_loop(0, D // 8, dbody, 0)
            return 0

        lax.fori_loop(0, rows_per_w // L, gbody, 0)

        pltpu.sync_copy(out_v, out_hbm.at[pl.ds(base * D, rows_per_w * D)])

    return sc_tem


def _tc_body(x_ref, w_ref, tem_ref, pe_ref, ent_ref, o_ref):
    _, Tt, F, A = x_ref.shape
    D = o_ref.shape[2]
    xm = x_ref[0].reshape(Tt * F, A)
    rbp = tem_ref[0] + pe_ref[...]
    rb = jnp.stack([rbp[:, :D], rbp[:, D:]], axis=1).reshape(Tt, D)
    rbb = lax.broadcast_in_dim(rb, (Tt, D, A), (0, 1))
    entb = ent_ref[...][None]
    G = 4 * F
    for j in range(Tt // 4):
        oj = jnp.dot(
            w_ref[...], xm[j * G:(j + 1) * G],
            preferred_element_type=jnp.float32,
        )
        o_ref[0, 4 * j:4 * j + 4] = (
            oj.reshape(4, D, A) + rbb[4 * j:4 * j + 4] + entb
        )


def _tc_fused(xT, W_bd, tem3, pe2, entT):
    B, T, F, A = xT.shape
    D = W_bd.shape[0] // 4
    TD = tem3.shape[1]
    return pl.pallas_call(
        _tc_body,
        grid=(B,),
        in_specs=[
            pl.BlockSpec((1, T, F, A), lambda b: (b, 0, 0, 0)),
            pl.BlockSpec((4 * D, 4 * F), lambda b: (0, 0)),
            pl.BlockSpec((1, TD, 128), lambda b: (b, 0, 0)),
            pl.BlockSpec((TD, 128), lambda b: (0, 0)),
            pl.BlockSpec((D, A), lambda b: (0, 0)),
        ],
        out_specs=pl.BlockSpec((1, T, D, A), lambda b: (b, 0, 0, 0)),
        out_shape=jax.ShapeDtypeStruct((B, T, D, A), jnp.float32),
    )(xT, W_bd, tem3, pe2, entT)


def kernel(x, x_temp, W, b, t_month, t_weekday, t_hour, t_year, ent_table):
    B, T, A, F = x.shape
    D = W.shape[1]
    BT = B * T

    xtn = jnp.transpose(x_temp.astype(jnp.int32), (0, 2, 1)).reshape(-1)

    sc_tem = _make_sc_tem(
        BT, T, D,
        t_month.shape[0], t_weekday.shape[0], t_hour.shape[0], t_year.shape[0],
    )
    tab_cat = jnp.concatenate(
        [t_month.reshape(-1), t_weekday.reshape(-1), t_hour.reshape(-1),
         t_year.reshape(-1)]
    )
    tem_flat = sc_tem(xtn, tab_cat)

    pos = jnp.arange(T, dtype=jnp.float32)[:, None]
    div = jnp.exp(
        jnp.arange(0, D, 2, dtype=jnp.float32) * (-math.log(10000.0) / D)
    )
    ang = pos * div[None, :]
    pe = jnp.stack([jnp.sin(ang), jnp.cos(ang)], axis=-1).reshape(T, D)
    pe_b = pe + b[None, :]
    pe2 = pe_b.reshape(T * D // 128, 128)
    tem3 = tem_flat.reshape(B, T * D // 128, 128)

    xT = jnp.transpose(x, (0, 1, 3, 2))
    W_bd = jnp.kron(jnp.eye(4, dtype=jnp.float32), W.T)
    entT = ent_table.T
    outT = _tc_fused(xT, W_bd, tem3, pe2, entT)
    return jnp.transpose(outT, (0, 1, 3, 2))


# speedup vs baseline: 3.9216x; 1.0265x over previous
import functools
import math

import jax
import jax.numpy as jnp
from jax import lax
from jax.experimental import pallas as pl
from jax.experimental.pallas import tpu as pltpu
from jax.experimental.pallas import tpu_sc as plsc

def _make_sc_tem(BT, T, D, n_month, n_weekday, n_hour, n_year):
    mesh = plsc.VectorSubcoreMesh(core_axis_name="c", subcore_axis_name="s")
    NC = mesh.num_cores
    NS = mesh.num_subcores
    NW = NC * NS
    assert BT % NW == 0
    rows_per_w = BT // NW
    assert rows_per_w % 16 == 0 and T % rows_per_w == 0
    L = 16

    n_tab = n_month + n_weekday + n_hour + n_year
    wd_base = n_month * D
    hr_base = (n_month + n_weekday) * D
    yr_base = (n_month + n_weekday + n_hour) * D

    @functools.partial(
        pl.kernel,
        out_type=jax.ShapeDtypeStruct((BT * D,), jnp.float32),
        mesh=mesh,
        compiler_params=pltpu.CompilerParams(needs_layout_passes=False),
        scratch_types=[
            pltpu.VMEM((rows_per_w * 4,), jnp.int32),
            pltpu.VMEM((n_tab * D,), jnp.float32),
            pltpu.VMEM((rows_per_w * D,), jnp.float32),
            pltpu.SemaphoreType.DMA,
        ],
    )
    def sc_tem(xt_hbm, tab_hbm, out_hbm, idx_v, tab_v, out_v, sem):
        wid = lax.axis_index("s") * NC + lax.axis_index("c")
        base = wid * rows_per_w
        b_i = base // T
        t0 = jnp.remainder(base, T)
        copies = [
            pltpu.make_async_copy(
                xt_hbm.at[pl.ds(b_i * 4 * T + k * T + t0, rows_per_w)],
                idx_v.at[pl.ds(k * rows_per_w, rows_per_w)],
                sem,
            )
            for k in range(4)
        ] + [pltpu.make_async_copy(tab_hbm, tab_v, sem)]
        for c in copies:
            c.start()
        for c in copies:
            c.wait()

        lanes = lax.iota(jnp.int32, L)

        def gbody(g, _):
            rows_loc = lanes + g * L
            m_off = plsc.load_gather(idx_v, [rows_loc]) * D
            w_off = plsc.load_gather(idx_v, [rows_loc + rows_per_w]) * D + wd_base
            h_off = plsc.load_gather(idx_v, [rows_loc + 2 * rows_per_w]) * D + hr_base
            y_idx = jnp.minimum(
                plsc.load_gather(idx_v, [rows_loc + 3 * rows_per_w]),
                n_year - 1,
            )
            y_off = y_idx * D + yr_base
            r_off = rows_loc * D

            def dbody(dc, _):
                for u in range(8):
                    dcol = jnp.bitwise_and(dc * 8 + u + lanes, D - 1)
                    acc = plsc.load_gather(tab_v, [m_off + dcol])
                    acc = acc + plsc.load_gather(tab_v, [w_off + dcol])
                    acc = acc + plsc.load_gather(tab_v, [h_off + dcol])
                    acc = acc + plsc.load_gather(tab_v, [y_off + dcol])
                    plsc.store_scatter(out_v, [r_off + dcol], acc)
                return 0

            lax.fori_loop(0, D // 8, dbody, 0)
            return 0

        lax.fori_loop(0, rows_per_w // L, gbody, 0)

        pltpu.sync_copy(out_v, out_hbm.at[pl.ds(base * D, rows_per_w * D)])

    return sc_tem


def _tc_body(x_ref, w_ref, tem_ref, pe_ref, ent_ref, o_ref):
    _, Tt, F, A = x_ref.shape
    D = o_ref.shape[2]
    xm = x_ref[0].reshape(Tt * F, A)
    rbp = tem_ref[0] + pe_ref[...]
    rb = jnp.stack([rbp[:, :D], rbp[:, D:]], axis=1).reshape(Tt, D)
    rbb = lax.broadcast_in_dim(rb, (Tt, D, A), (0, 1))
    entb = ent_ref[...][None]
    G = 4 * F
    for j in range(Tt // 4):
        oj = jnp.dot(
            w_ref[...], xm[j * G:(j + 1) * G],
            preferred_element_type=jnp.float32,
        )
        o_ref[0, 4 * j:4 * j + 4] = (
            oj.reshape(4, D, A) + rbb[4 * j:4 * j + 4] + entb
        )


def _tc_fused(xT, W_bd, tem3, pe2, entT):
    B, T, F, A = xT.shape
    D = W_bd.shape[0] // 4
    TD = tem3.shape[1]
    return pl.pallas_call(
        _tc_body,
        grid=(B,),
        in_specs=[
            pl.BlockSpec((1, T, F, A), lambda b: (b, 0, 0, 0)),
            pl.BlockSpec((4 * D, 4 * F), lambda b: (0, 0)),
            pl.BlockSpec((1, TD, 128), lambda b: (b, 0, 0)),
            pl.BlockSpec((TD, 128), lambda b: (0, 0)),
            pl.BlockSpec((D, A), lambda b: (0, 0)),
        ],
        out_specs=pl.BlockSpec((1, T, D, A), lambda b: (b, 0, 0, 0)),
        out_shape=jax.ShapeDtypeStruct((B, T, D, A), jnp.float32),
    )(xT, W_bd, tem3, pe2, entT)


def kernel(x, x_temp, W, b, t_month, t_weekday, t_hour, t_year, ent_table):
    B, T, A, F = x.shape
    D = W.shape[1]
    BT = B * T

    xtn = jnp.transpose(x_temp.astype(jnp.int32), (0, 2, 1)).reshape(-1)

    sc_tem = _make_sc_tem(
        BT, T, D,
        t_month.shape[0], t_weekday.shape[0], t_hour.shape[0], t_year.shape[0],
    )
    tab_cat = jnp.concatenate(
        [t_month, t_weekday, t_hour, t_year], axis=0
    ).reshape(-1)
    tem_flat = sc_tem(xtn, tab_cat)

    TD = T * D // 128
    div = jnp.exp(
        jnp.arange(0, D, 2, dtype=jnp.float32) * (-math.log(10000.0) / D)
    )
    i2 = jnp.arange(TD, dtype=jnp.int32)[:, None]
    l2 = jnp.arange(128, dtype=jnp.int32)[None, :]
    tt = (2 * i2 + (l2 >= D).astype(jnp.int32)).astype(jnp.float32)
    dd = jnp.bitwise_and(l2, D - 1)
    ang = tt * jnp.take(div, dd >> 1)
    pe2v = jnp.where(jnp.bitwise_and(dd, 1) == 1, jnp.cos(ang), jnp.sin(ang))
    pe2 = pe2v + jnp.concatenate([b, b])[None, :]
    tem3 = tem_flat.reshape(B, TD, 128)

    xT = jnp.transpose(x, (0, 1, 3, 2))
    W_bd = jnp.kron(jnp.eye(4, dtype=jnp.float32), W.T)
    entT = ent_table.T
    outT = _tc_fused(xT, W_bd, tem3, pe2, entT)
    return jnp.transpose(outT, (0, 1, 3, 2))
